# Initial kernel scaffold; baseline (speedup 1.0000x reference)
import jax
import jax.numpy as jnp
from jax.experimental import pallas as pl


def kernel(logits, labels):
    raise NotImplementedError("write your pallas kernel here")


# speedup vs baseline: 14.4759x; 14.4759x over previous
import jax
import jax.numpy as jnp
from jax.experimental import pallas as pl
from jax.experimental.pallas import tpu as pltpu

_N = 8 * 512 * 512
_K = _N // 4
_R = 4096
_C = 512
_CHUNK = 256
_NCH = _R // _CHUNK
_HI0 = 0x7F800001


def _topk_mean_body(x_ref, y_ref, o_ref, bits_ref):
    def compute_chunk(i, carry):
        sl = pl.ds(i * _CHUNK, _CHUNK)
        x = x_ref[sl, :]
        y = y_ref[sl, :]
        loss = jnp.maximum(x, 0.0) - x * y + jnp.log1p(jnp.exp(-jnp.abs(x)))
        bits_ref[sl, :] = jax.lax.bitcast_convert_type(loss, jnp.int32)
        return carry

    jax.lax.fori_loop(0, _NCH, compute_chunk, 0)

    def count_gt(t):
        def cc(i, acc):
            b = bits_ref[pl.ds(i * _CHUNK, _CHUNK), :]
            return acc + jnp.sum((b > t).astype(jnp.int32))

        return jax.lax.fori_loop(0, _NCH, cc, jnp.int32(0))

    def bisect(_, carry):
        lo, hi, c_hi = carry
        mid = lo + (hi - lo) // 2
        c = count_gt(mid)
        pred = c >= _K
        return (
            jnp.where(pred, mid, lo),
            jnp.where(pred, hi, mid),
            jnp.where(pred, c_hi, c),
        )

    lo, hi, c_hi = jax.lax.fori_loop(
        0,
        31,
        bisect,
        (jnp.int32(-1), jnp.int32(_HI0), jnp.int32(0)),
    )

    t = hi
    t_val = jax.lax.bitcast_convert_type(t, jnp.float32)

    def sum_gt(i, acc):
        b = bits_ref[pl.ds(i * _CHUNK, _CHUNK), :]
        v = jax.lax.bitcast_convert_type(b, jnp.float32)
        return acc + jnp.sum(jnp.where(b > t, v, 0.0))

    s = jax.lax.fori_loop(0, _NCH, sum_gt, jnp.float32(0.0))

    mean = (s + (_K - c_hi).astype(jnp.float32) * t_val) / jnp.float32(_K)
    o_ref[0, 0] = mean


def kernel(logits, labels):
    x = logits.reshape(_R, _C)
    y = labels.astype(jnp.float32).reshape(_R, _C)
    out = pl.pallas_call(
        _topk_mean_body,
        out_shape=jax.ShapeDtypeStruct((1, 1), jnp.float32),
        out_specs=pl.BlockSpec(memory_space=pltpu.SMEM),
        scratch_shapes=[pltpu.VMEM((_R, _C), jnp.int32)],
    )(x, y)
    return out[0, 0]
